# baseline (speedup 1.0000x reference)
import jax
import jax.numpy as jnp
from jax.experimental import pallas as pl
from jax.experimental.pallas import tpu as pltpu

_NCHUNKS = 8


def _dma_copy(x_ref, o_ref, sems):
    rows = x_ref.shape[0] // _NCHUNKS
    for i in range(_NCHUNKS):
        pltpu.make_async_copy(
            x_ref.at[pl.ds(i * rows, rows)],
            o_ref.at[pl.ds(i * rows, rows)],
            sems.at[i],
        ).start()
    for i in range(_NCHUNKS):
        pltpu.make_async_copy(
            x_ref.at[pl.ds(i * rows, rows)],
            o_ref.at[pl.ds(i * rows, rows)],
            sems.at[i],
        ).wait()


def kernel(rays):
    return pl.pallas_call(
        _dma_copy,
        in_specs=[pl.BlockSpec(memory_space=pl.ANY)],
        out_specs=pl.BlockSpec(memory_space=pl.ANY),
        out_shape=jax.ShapeDtypeStruct(rays.shape, rays.dtype),
        scratch_shapes=[pltpu.SemaphoreType.DMA((_NCHUNKS,))],
    )(rays)


# speedup vs baseline: 18.6883x; 18.6883x over previous
import jax
import jax.numpy as jnp
from jax.experimental import pallas as pl


def _copy_block(x_ref, o_ref):
    o_ref[...] = x_ref[...]


def kernel(rays):
    n, d = rays.shape
    flat = rays.reshape(-1, 128)
    rows = flat.shape[0]
    block_rows = 8192
    grid = rows // block_rows
    out = pl.pallas_call(
        _copy_block,
        grid=(grid,),
        in_specs=[pl.BlockSpec((block_rows, 128), lambda i: (i, 0))],
        out_specs=pl.BlockSpec((block_rows, 128), lambda i: (i, 0)),
        out_shape=jax.ShapeDtypeStruct(flat.shape, flat.dtype),
    )(flat)
    return out.reshape(n, d)


# speedup vs baseline: 794.4842x; 42.5124x over previous
import jax
import jax.numpy as jnp
from jax.experimental import pallas as pl


def _copy_block(x_ref, o_ref):
    o_ref[...] = x_ref[...]


def kernel(rays):
    n, d = rays.shape
    t = rays.T
    block_l = 131072
    grid = n // block_l
    out = pl.pallas_call(
        _copy_block,
        grid=(grid,),
        in_specs=[pl.BlockSpec((d, block_l), lambda i: (0, i))],
        out_specs=pl.BlockSpec((d, block_l), lambda i: (0, i)),
        out_shape=jax.ShapeDtypeStruct(t.shape, t.dtype),
    )(t)
    return out.T


# speedup vs baseline: 811.0249x; 1.0208x over previous
import jax
import jax.numpy as jnp
from jax.experimental import pallas as pl


def _copy_block(x_ref, o_ref):
    o_ref[...] = x_ref[...]


def kernel(rays):
    n, d = rays.shape
    t = rays.T
    block_l = 262144
    grid = n // block_l
    out = pl.pallas_call(
        _copy_block,
        grid=(grid,),
        in_specs=[pl.BlockSpec((d, block_l), lambda i: (0, i))],
        out_specs=pl.BlockSpec((d, block_l), lambda i: (0, i)),
        out_shape=jax.ShapeDtypeStruct(t.shape, t.dtype),
    )(t)
    return out.T
